# baseline (speedup 1.0000x reference)
import jax
import jax.numpy as jnp
from jax.experimental import pallas as pl
from jax.experimental.pallas import tpu as pltpu

_BM = 512
_SENTINEL = 384.0


def _count_below(arr_ref, tbf):
    rows, _ = arr_ref.shape
    chunk = 256
    total = jnp.float32(0)
    for r in range(rows // chunk):
        cs = jnp.sum(
            (arr_ref[pl.ds(r * chunk, chunk), :] < tbf).astype(jnp.bfloat16),
            axis=0)
        total = total + jnp.sum(cs.astype(jnp.float32))
    return total.astype(jnp.int32)


def _radix_phase(arr_ref, nbits, k_sub):
    def step(j, carry):
        prefix, cb = carry
        trial = prefix + jax.lax.shift_left(jnp.int32(1),
                                            jnp.int32(nbits - 1) - j)
        c = _count_below(arr_ref, trial.astype(jnp.bfloat16))
        take = c <= k_sub
        return (jnp.where(take, trial, prefix), jnp.where(take, c, cb))

    return jax.lax.fori_loop(0, nbits, step,
                             (jnp.int32(0), jnp.int32(0)))


def _fused_body(x_ref, w_ref, b_ref, out_ref, arr_ref, wt_ref, k_rank):
    i = pl.program_id(0)
    n_out, n_in = w_ref.shape
    tile = 256

    def _abs_bits(rs):
        return jax.lax.bitcast_convert_type(
            w_ref[rs, :], jnp.int32) & jnp.int32(0x7FFFFFFF)

    @pl.when(i == 0)
    def _select_and_mask():
        for r in range(n_out // tile):
            rs = pl.ds(r * tile, tile)
            arr_ref[rs, :] = jax.lax.shift_right_logical(
                _abs_bits(rs), jnp.int32(23)).astype(jnp.bfloat16)

        f1, cb1 = _radix_phase(arr_ref, 8, k_rank)
        k2 = k_rank - cb1

        def refold(match_val, shift, width):
            mbf = match_val.astype(jnp.bfloat16)
            for r in range(n_out // tile):
                rs = pl.ds(r * tile, tile)
                field = (jax.lax.shift_right_logical(
                    _abs_bits(rs), jnp.int32(shift))
                         & jnp.int32((1 << width) - 1)).astype(jnp.bfloat16)
                arr_ref[rs, :] = jnp.where(arr_ref[rs, :] == mbf, field,
                                           jnp.bfloat16(_SENTINEL))

        refold(f1, 15, 8)
        f2, cb2 = _radix_phase(arr_ref, 8, k2)
        k3 = k2 - cb2

        refold(f2, 7, 8)
        f3, cb3 = _radix_phase(arr_ref, 8, k3)
        k4 = k3 - cb3

        refold(f3, 0, 7)
        f4, _ = _radix_phase(arr_ref, 7, k4)

        tbits = (jax.lax.shift_left(f1, jnp.int32(23))
                 | jax.lax.shift_left(f2, jnp.int32(15))
                 | jax.lax.shift_left(f3, jnp.int32(7)) | f4)
        t = jax.lax.bitcast_convert_type(tbits, jnp.float32)
        for ti in range(n_out // tile):
            for tj in range(n_in // tile):
                wtile = w_ref[pl.ds(ti * tile, tile), pl.ds(tj * tile, tile)]
                wm = jnp.where(jnp.abs(wtile) > t, wtile, 0.0)
                wt_ref[pl.ds(tj * tile, tile), pl.ds(ti * tile, tile)] = (
                    wm.T.astype(jnp.bfloat16))

    @pl.when(i > 0)
    def _gemm():
        xb = x_ref[...].astype(jnp.bfloat16)
        acc = jnp.dot(xb, wt_ref[...], preferred_element_type=jnp.float32)
        out_ref[...] = acc + b_ref[...]


def kernel(input, weight, bias):
    n_out, n_in = weight.shape
    x2d = input.reshape(-1, n_in)
    m = x2d.shape[0]
    k_rank = (n_out * n_in) // 2 - 1

    out = pl.pallas_call(
        lambda x_ref, w_ref, b_ref, out_ref, arr_ref, wt_ref: _fused_body(
            x_ref, w_ref, b_ref, out_ref, arr_ref, wt_ref, k_rank),
        grid=(1 + m // _BM,),
        in_specs=[
            pl.BlockSpec((_BM, n_in), lambda i: (jnp.maximum(i - 1, 0), 0)),
            pl.BlockSpec((n_out, n_in), lambda i: (0, 0)),
            pl.BlockSpec((1, n_out), lambda i: (0, 0)),
        ],
        out_specs=pl.BlockSpec((_BM, n_out),
                               lambda i: (jnp.maximum(i - 1, 0), 0)),
        out_shape=jax.ShapeDtypeStruct((m, n_out), jnp.float32),
        scratch_shapes=[
            pltpu.VMEM((n_out, n_in), jnp.bfloat16),
            pltpu.VMEM((n_in, n_out), jnp.bfloat16),
        ],
    )(x2d, weight, bias.reshape(1, n_out))

    return out.reshape(*input.shape[:-1], n_out)


# speedup vs baseline: 1.8051x; 1.8051x over previous
import jax
import jax.numpy as jnp
from jax.experimental import pallas as pl
from jax.experimental.pallas import tpu as pltpu

_BM = 512


def _count_below(w_ref, trial):
    n_out, n_in = w_ref.shape
    chunk = 256
    ones = jnp.ones((1, chunk), jnp.float32)
    acc = jnp.zeros((1, n_in), jnp.float32)
    for r in range(n_out // chunk):
        bits = jax.lax.bitcast_convert_type(
            w_ref[pl.ds(r * chunk, chunk), :],
            jnp.int32) & jnp.int32(0x7FFFFFFF)
        condf = (bits < trial).astype(jnp.float32)
        acc = acc + jnp.dot(ones, condf, preferred_element_type=jnp.float32)
    return jnp.sum(acc).astype(jnp.int32)


def _fused_body(x_ref, w_ref, b_ref, out_ref, wt_ref, k_rank):
    i = pl.program_id(0)
    n_out, n_in = w_ref.shape
    tile = 256

    @pl.when(i == 0)
    def _select_and_mask():
        def step(j, prefix):
            trial = prefix + jax.lax.shift_left(jnp.int32(1),
                                                jnp.int32(30) - j)
            c = _count_below(w_ref, trial)
            return jnp.where(c <= k_rank, trial, prefix)

        tbits = jax.lax.fori_loop(0, 31, step, jnp.int32(0))
        t = jax.lax.bitcast_convert_type(tbits, jnp.float32)
        for ti in range(n_out // tile):
            for tj in range(n_in // tile):
                wtile = w_ref[pl.ds(ti * tile, tile), pl.ds(tj * tile, tile)]
                wm = jnp.where(jnp.abs(wtile) > t, wtile, 0.0)
                wt_ref[pl.ds(tj * tile, tile), pl.ds(ti * tile, tile)] = (
                    wm.T.astype(jnp.bfloat16))

    @pl.when(i > 0)
    def _gemm():
        xb = x_ref[...].astype(jnp.bfloat16)
        acc = jnp.dot(xb, wt_ref[...], preferred_element_type=jnp.float32)
        out_ref[...] = acc + b_ref[...]


def kernel(input, weight, bias):
    n_out, n_in = weight.shape
    x2d = input.reshape(-1, n_in)
    m = x2d.shape[0]
    k_rank = (n_out * n_in) // 2 - 1

    out = pl.pallas_call(
        lambda x_ref, w_ref, b_ref, out_ref, wt_ref: _fused_body(
            x_ref, w_ref, b_ref, out_ref, wt_ref, k_rank),
        grid=(1 + m // _BM,),
        in_specs=[
            pl.BlockSpec((_BM, n_in), lambda i: (jnp.maximum(i - 1, 0), 0)),
            pl.BlockSpec((n_out, n_in), lambda i: (0, 0)),
            pl.BlockSpec((1, n_out), lambda i: (0, 0)),
        ],
        out_specs=pl.BlockSpec((_BM, n_out),
                               lambda i: (jnp.maximum(i - 1, 0), 0)),
        out_shape=jax.ShapeDtypeStruct((m, n_out), jnp.float32),
        scratch_shapes=[
            pltpu.VMEM((n_in, n_out), jnp.bfloat16),
        ],
    )(x2d, weight, bias.reshape(1, n_out))

    return out.reshape(*input.shape[:-1], n_out)


# speedup vs baseline: 1.9035x; 1.0545x over previous
import jax
import jax.numpy as jnp
from jax.experimental import pallas as pl
from jax.experimental.pallas import tpu as pltpu

_BM = 512


def _count_below(w_ref, trial):
    n_out, n_in = w_ref.shape
    chunk = 256
    ones = jnp.ones((1, chunk), jnp.float32)
    acc = jnp.zeros((1, n_in), jnp.float32)
    for r in range(n_out // chunk):
        bits = jax.lax.bitcast_convert_type(
            w_ref[pl.ds(r * chunk, chunk), :],
            jnp.int32) & jnp.int32(0x7FFFFFFF)
        condf = (bits < trial).astype(jnp.float32)
        acc = acc + jnp.dot(ones, condf, preferred_element_type=jnp.float32)
    return jnp.sum(acc).astype(jnp.int32)


def _fused_body(x_ref, w_ref, b_ref, out_ref, wt_ref, k_rank):
    i = pl.program_id(0)
    n_out, n_in = w_ref.shape
    tile = 256

    @pl.when(i == 0)
    def _select_and_mask():
        def cond_fn(carry):
            j, _, found = carry
            return jnp.logical_and(j < 31, jnp.logical_not(found))

        def body_fn(carry):
            j, prefix, _ = carry
            trial = prefix + jax.lax.shift_left(jnp.int32(1),
                                                jnp.int32(30) - j)
            c = _count_below(w_ref, trial)
            found = c == k_rank + 1
            nxt = jnp.where(found, trial - 1,
                            jnp.where(c <= k_rank, trial, prefix))
            return (j + 1, nxt, found)

        _, tbits, _ = jax.lax.while_loop(
            cond_fn, body_fn, (jnp.int32(0), jnp.int32(0), False))
        t = jax.lax.bitcast_convert_type(tbits, jnp.float32)
        for ti in range(n_out // tile):
            for tj in range(n_in // tile):
                wtile = w_ref[pl.ds(ti * tile, tile), pl.ds(tj * tile, tile)]
                wm = jnp.where(jnp.abs(wtile) > t, wtile, 0.0)
                wt_ref[pl.ds(tj * tile, tile), pl.ds(ti * tile, tile)] = (
                    wm.T.astype(jnp.bfloat16))

    @pl.when(i > 0)
    def _gemm():
        xb = x_ref[...].astype(jnp.bfloat16)
        acc = jnp.dot(xb, wt_ref[...], preferred_element_type=jnp.float32)
        out_ref[...] = acc + b_ref[...]


def kernel(input, weight, bias):
    n_out, n_in = weight.shape
    x2d = input.reshape(-1, n_in)
    m = x2d.shape[0]
    k_rank = (n_out * n_in) // 2 - 1

    out = pl.pallas_call(
        lambda x_ref, w_ref, b_ref, out_ref, wt_ref: _fused_body(
            x_ref, w_ref, b_ref, out_ref, wt_ref, k_rank),
        grid=(1 + m // _BM,),
        in_specs=[
            pl.BlockSpec((_BM, n_in), lambda i: (jnp.maximum(i - 1, 0), 0)),
            pl.BlockSpec((n_out, n_in), lambda i: (0, 0)),
            pl.BlockSpec((1, n_out), lambda i: (0, 0)),
        ],
        out_specs=pl.BlockSpec((_BM, n_out),
                               lambda i: (jnp.maximum(i - 1, 0), 0)),
        out_shape=jax.ShapeDtypeStruct((m, n_out), jnp.float32),
        scratch_shapes=[
            pltpu.VMEM((n_in, n_out), jnp.bfloat16),
        ],
    )(x2d, weight, bias.reshape(1, n_out))

    return out.reshape(*input.shape[:-1], n_out)
